# baseline (speedup 1.0000x reference)
import functools

import jax
import jax.numpy as jnp
from jax import lax
from jax.experimental import pallas as pl
from jax.experimental.pallas import tpu as pltpu
from jax.experimental.pallas import tpu_sc as plsc

C_IN = 100000
D_MODEL = 64

NC = 2
NS = 16
NW = NC * NS
LANE = 128
DT = D_MODEL // 8


def _make_gather(batch: int, seq: int):
    assert batch == NW * LANE and seq % 2 == 0
    mesh = plsc.VectorSubcoreMesh(core_axis_name="c", subcore_axis_name="s")

    @functools.partial(
        pl.kernel,
        out_type=jax.ShapeDtypeStruct((seq, DT, NW, 8, LANE), jnp.float32),
        mesh=mesh,
        scratch_types=[
            pltpu.VMEM((seq, LANE), jnp.int32),
            pltpu.VMEM((LANE, D_MODEL), jnp.float32),
            pltpu.VMEM((LANE, D_MODEL), jnp.float32),
            pltpu.VMEM((D_MODEL, LANE), jnp.float32),
            pltpu.VMEM((D_MODEL, LANE), jnp.float32),
            pltpu.SemaphoreType.DMA,
            pltpu.SemaphoreType.DMA,
            pltpu.SemaphoreType.DMA,
            pltpu.SemaphoreType.DMA,
        ],
        compiler_params=pltpu.CompilerParams(use_tc_tiling_on_sc=False, needs_layout_passes=False),
    )
    def gather_kernel(w_hbm, xt_hbm, out_hbm, idx_v, ga, gb, ta, tb,
                      gsem_a, gsem_b, wsem_a, wsem_b):
        wid = lax.axis_index("s") * NC + lax.axis_index("c")
        pltpu.sync_copy(xt_hbm.at[:, pl.ds(wid * LANE, LANE)], idx_v)

        iota16 = lax.iota(jnp.int32, 16)
        bidx = [j * 16 + iota16 for j in range(LANE // 16)]

        def start_gather(s, gbuf, gsem):
            pltpu.async_copy(w_hbm.at[idx_v.at[s]], gbuf, gsem)

        def drain_gather(gbuf, gsem):
            pltpu.make_async_copy(w_hbm.at[idx_v.at[0]], gbuf, gsem).wait()

        def transpose(gbuf, tbuf):
            @plsc.parallel_loop(0, D_MODEL, unroll=8)
            def d_body(d):
                didx = jnp.full((16,), d, jnp.int32)
                for j in range(LANE // 16):
                    v = plsc.load_gather(gbuf, [bidx[j], didx])
                    tbuf[d, pl.ds(j * 16, 16)] = v

        def start_write(s, tbuf, wsem):
            for dt in range(DT):
                pltpu.async_copy(tbuf.at[pl.ds(dt * 8, 8)],
                                 out_hbm.at[s, dt, wid], wsem)

        def drain_write(tbuf, wsem):
            for dt in range(DT):
                pltpu.make_async_copy(tbuf.at[pl.ds(dt * 8, 8)],
                                      out_hbm.at[0, dt, wid], wsem).wait()

        def step(s, gbuf, gsem, nxt_gbuf, nxt_gsem, tbuf, wsem):
            drain_gather(gbuf, gsem)

            @pl.when(s + 1 < seq)
            def _():
                start_gather(s + 1, nxt_gbuf, nxt_gsem)

            @pl.when(s >= 2)
            def _():
                drain_write(tbuf, wsem)

            transpose(gbuf, tbuf)
            start_write(s, tbuf, wsem)

        start_gather(0, ga, gsem_a)

        def pair_body(t, carry):
            s = 2 * t
            step(s, ga, gsem_a, gb, gsem_b, ta, wsem_a)
            step(s + 1, gb, gsem_b, ga, gsem_a, tb, wsem_b)
            return carry

        lax.fori_loop(0, seq // 2, pair_body, 0)
        drain_write(ta, wsem_a)
        drain_write(tb, wsem_b)

    return gather_kernel


def kernel(x, W):
    b, s = x.shape
    xt = x.astype(jnp.int32).T
    out5 = _make_gather(b, s)(W, xt)
    return out5.transpose(2, 4, 0, 1, 3).reshape(b, s, D_MODEL)


# speedup vs baseline: 2.4758x; 2.4758x over previous
import functools

import jax
import jax.numpy as jnp
from jax import lax
from jax.experimental import pallas as pl
from jax.experimental.pallas import tpu as pltpu
from jax.experimental.pallas import tpu_sc as plsc

C_IN = 100000
D_MODEL = 64

NC = 2
NS = 16
NW = NC * NS
LANE = 128
DT = D_MODEL // 8


def _make_gather(batch: int, seq: int):
    assert batch == NW * LANE and seq % 2 == 0
    mesh = plsc.VectorSubcoreMesh(core_axis_name="c", subcore_axis_name="s")

    @functools.partial(
        pl.kernel,
        out_type=jax.ShapeDtypeStruct((seq, DT, NW, 8, LANE), jnp.float32),
        mesh=mesh,
        scratch_types=[
            pltpu.VMEM((seq, LANE), jnp.int32),
            pltpu.VMEM((LANE, D_MODEL), jnp.float32),
            pltpu.VMEM((LANE, D_MODEL), jnp.float32),
            pltpu.VMEM((D_MODEL, LANE + 1), jnp.float32),
            pltpu.VMEM((D_MODEL, LANE + 1), jnp.float32),
            pltpu.SemaphoreType.DMA,
            pltpu.SemaphoreType.DMA,
            pltpu.SemaphoreType.DMA,
            pltpu.SemaphoreType.DMA,
        ],
        compiler_params=pltpu.CompilerParams(use_tc_tiling_on_sc=False, needs_layout_passes=False),
    )
    def gather_kernel(w_hbm, xt_hbm, out_hbm, idx_v, ga, gb, ta, tb,
                      gsem_a, gsem_b, wsem_a, wsem_b):
        wid = lax.axis_index("s") * NC + lax.axis_index("c")
        pltpu.sync_copy(xt_hbm.at[:, pl.ds(wid * LANE, LANE)], idx_v)

        iota16 = lax.iota(jnp.int32, 16)
        ridx = [j * 16 + iota16 for j in range(D_MODEL // 16)]

        def start_gather(s, gbuf, gsem):
            pltpu.async_copy(w_hbm.at[idx_v.at[s]], gbuf, gsem)

        def drain_gather(gbuf, gsem):
            pltpu.make_async_copy(w_hbm.at[idx_v.at[0]], gbuf, gsem).wait()

        def transpose(gbuf, tbuf):
            @plsc.parallel_loop(0, LANE, unroll=8)
            def b_body(b):
                cidx = jnp.full((16,), b, jnp.int32)
                for j in range(D_MODEL // 16):
                    v = gbuf[b, pl.ds(j * 16, 16)]
                    plsc.store_scatter(tbuf, [ridx[j], cidx], v)

        def start_write(s, tbuf, wsem):
            for dt in range(DT):
                pltpu.async_copy(tbuf.at[pl.ds(dt * 8, 8), pl.ds(0, LANE)],
                                 out_hbm.at[s, dt, wid], wsem)

        def drain_write(tbuf, wsem):
            for dt in range(DT):
                pltpu.make_async_copy(tbuf.at[pl.ds(dt * 8, 8), pl.ds(0, LANE)],
                                      out_hbm.at[0, dt, wid], wsem).wait()

        def step(s, gbuf, gsem, nxt_gbuf, nxt_gsem, tbuf, wsem):
            drain_gather(gbuf, gsem)

            @pl.when(s + 1 < seq)
            def _():
                start_gather(s + 1, nxt_gbuf, nxt_gsem)

            @pl.when(s >= 2)
            def _():
                drain_write(tbuf, wsem)

            transpose(gbuf, tbuf)
            start_write(s, tbuf, wsem)

        start_gather(0, ga, gsem_a)

        def pair_body(t, carry):
            s = 2 * t
            step(s, ga, gsem_a, gb, gsem_b, ta, wsem_a)
            step(s + 1, gb, gsem_b, ga, gsem_a, tb, wsem_b)
            return carry

        lax.fori_loop(0, seq // 2, pair_body, 0)
        drain_write(ta, wsem_a)
        drain_write(tb, wsem_b)

    return gather_kernel


def kernel(x, W):
    b, s = x.shape
    xt = x.astype(jnp.int32).T
    out5 = _make_gather(b, s)(W, xt)
    return out5.transpose(2, 4, 0, 1, 3).reshape(b, s, D_MODEL)


# speedup vs baseline: 3.3228x; 1.3421x over previous
import functools

import jax
import jax.numpy as jnp
from jax import lax
from jax.experimental import pallas as pl
from jax.experimental.pallas import tpu as pltpu
from jax.experimental.pallas import tpu_sc as plsc

C_IN = 100000
D_MODEL = 64

NC = 2
NS = 16
NW = NC * NS
LANE = 128
DT = D_MODEL // 8


def _make_gather(batch: int, seq: int):
    assert batch == NW * LANE and seq % 4 == 0
    mesh = plsc.VectorSubcoreMesh(core_axis_name="c", subcore_axis_name="s")

    @functools.partial(
        pl.kernel,
        out_type=jax.ShapeDtypeStruct((seq, DT, NW, 8, LANE), jnp.float32),
        mesh=mesh,
        scratch_types=[
            pltpu.VMEM((seq, LANE), jnp.int32),
            pltpu.VMEM((LANE, D_MODEL), jnp.float32),
            pltpu.VMEM((LANE, D_MODEL), jnp.float32),
            pltpu.VMEM((LANE, D_MODEL), jnp.float32),
            pltpu.VMEM((LANE, D_MODEL), jnp.float32),
            pltpu.VMEM((D_MODEL, LANE + 1), jnp.float32),
            pltpu.VMEM((D_MODEL, LANE + 1), jnp.float32),
            pltpu.SemaphoreType.DMA,
            pltpu.SemaphoreType.DMA,
            pltpu.SemaphoreType.DMA,
            pltpu.SemaphoreType.DMA,
            pltpu.SemaphoreType.DMA,
            pltpu.SemaphoreType.DMA,
        ],
        compiler_params=pltpu.CompilerParams(use_tc_tiling_on_sc=False, needs_layout_passes=False),
    )
    def gather_kernel(w_hbm, xt_hbm, out_hbm, idx_v, g0, g1, g2, g3, ta, tb,
                      gsem0, gsem1, gsem2, gsem3, wsem_a, wsem_b):
        wid = lax.axis_index("s") * NC + lax.axis_index("c")
        pltpu.sync_copy(xt_hbm.at[:, pl.ds(wid * LANE, LANE)], idx_v)

        iota16 = lax.iota(jnp.int32, 16)
        ridx = [j * 16 + iota16 for j in range(D_MODEL // 16)]

        def start_gather(s, gbuf, gsem):
            pltpu.async_copy(w_hbm.at[idx_v.at[s]], gbuf, gsem)

        def drain_gather(gbuf, gsem):
            pltpu.make_async_copy(w_hbm.at[idx_v.at[0]], gbuf, gsem).wait()

        def transpose(gbuf, tbuf):
            @plsc.parallel_loop(0, LANE, unroll=8)
            def b_body(b):
                cidx = jnp.full((16,), b, jnp.int32)
                for j in range(D_MODEL // 16):
                    v = gbuf[b, pl.ds(j * 16, 16)]
                    plsc.store_scatter(tbuf, [ridx[j], cidx], v)

        def start_write(s, tbuf, wsem):
            for dt in range(DT):
                pltpu.async_copy(tbuf.at[pl.ds(dt * 8, 8), pl.ds(0, LANE)],
                                 out_hbm.at[s, dt, wid], wsem)

        def drain_write(tbuf, wsem):
            for dt in range(DT):
                pltpu.make_async_copy(tbuf.at[pl.ds(dt * 8, 8), pl.ds(0, LANE)],
                                      out_hbm.at[0, dt, wid], wsem).wait()

        def step(s, gbuf, gsem, pf_gbuf, pf_gsem, tbuf, wsem):
            drain_gather(gbuf, gsem)

            @pl.when(s + 2 < seq)
            def _():
                start_gather(s + 2, pf_gbuf, pf_gsem)

            @pl.when(s >= 2)
            def _():
                drain_write(tbuf, wsem)

            transpose(gbuf, tbuf)
            start_write(s, tbuf, wsem)

        start_gather(0, g0, gsem0)
        start_gather(1, g1, gsem1)

        def quad_body(t, carry):
            s = 4 * t
            step(s, g0, gsem0, g2, gsem2, ta, wsem_a)
            step(s + 1, g1, gsem1, g3, gsem3, tb, wsem_b)
            step(s + 2, g2, gsem2, g0, gsem0, ta, wsem_a)
            step(s + 3, g3, gsem3, g1, gsem1, tb, wsem_b)
            return carry

        lax.fori_loop(0, seq // 4, quad_body, 0)
        drain_write(ta, wsem_a)
        drain_write(tb, wsem_b)

    return gather_kernel


def kernel(x, W):
    b, s = x.shape
    xt = x.astype(jnp.int32).T
    out5 = _make_gather(b, s)(W, xt)
    return out5.transpose(2, 4, 0, 1, 3).reshape(b, s, D_MODEL)
